# baseline (speedup 1.0000x reference)
import jax
import jax.numpy as jnp
from jax.experimental import pallas as pl
from jax.experimental.pallas import tpu as pltpu

_R = 200


def _fused_body(x_ref, w1t_ref, b1_ref, w2t_ref, b2_ref, ca_ref, cb_ref,
                o_ref, xp_ref, hp_ref):
    i = pl.program_id(0)
    nb = pl.num_programs(0) // 2

    @pl.when(i == 0)
    def _():
        xp = jnp.dot(x_ref[...], w1t_ref[...],
                     preferred_element_type=jnp.float32)
        xp_ref[...] = xp.astype(jnp.bfloat16)

    ca = ca_ref[...].astype(jnp.bfloat16)
    cb = cb_ref[...].astype(jnp.bfloat16)

    @pl.when(i < nb)
    def _():
        xp = xp_ref[...]
        for c, off in ((ca, 0), (cb, _R)):
            h = jax.lax.dot_general(
                c, xp, (((1,), (0,)), ((), ())),
                preferred_element_type=jnp.float32)
            h = jnp.maximum(h + b1_ref[...], 0.0)
            hp = jnp.dot(h.astype(jnp.bfloat16), w2t_ref[...],
                         preferred_element_type=jnp.float32)
            hp_ref[pl.ds(i * 2 * _R + off, _R), :] = hp

    @pl.when(i >= nb)
    def _():
        hp = hp_ref[...].astype(jnp.bfloat16)
        for c, off in ((ca, 0), (cb, _R)):
            o_ref[pl.ds(off, _R), :] = jax.lax.dot_general(
                c, hp, (((1,), (0,)), ((), ())),
                preferred_element_type=jnp.float32) + b2_ref[...]


def kernel(x, edge_index, C, W1, b1, W2, b2):
    del edge_index
    n, in_dim = x.shape
    hid = W1.shape[0]
    ncls = W2.shape[0]
    nb = n // (2 * _R)

    return pl.pallas_call(
        _fused_body,
        grid=(2 * nb,),
        in_specs=[
            pl.BlockSpec((n, in_dim), lambda i: (0, 0)),
            pl.BlockSpec((in_dim, hid), lambda i: (0, 0)),
            pl.BlockSpec((1, hid), lambda i: (0, 0)),
            pl.BlockSpec((hid, ncls), lambda i: (0, 0)),
            pl.BlockSpec((1, ncls), lambda i: (0, 0)),
            pl.BlockSpec((_R, n), lambda i: (2 * (i % (pl.num_programs(0) // 2)), 0)),
            pl.BlockSpec((_R, n), lambda i: (2 * (i % (pl.num_programs(0) // 2)) + 1, 0)),
        ],
        out_specs=pl.BlockSpec((2 * _R, ncls),
                               lambda i: (i % (pl.num_programs(0) // 2), 0)),
        out_shape=jax.ShapeDtypeStruct((n, ncls), jnp.float32),
        scratch_shapes=[
            pltpu.VMEM((n, hid), jnp.bfloat16),
            pltpu.VMEM((n, ncls), jnp.float32),
        ],
    )(x, W1.T, b1.reshape(1, hid), W2.T.astype(jnp.bfloat16),
      b2.reshape(1, ncls), C, C)
